# baseline (speedup 1.0000x reference)
import math

import jax
import jax.numpy as jnp
from jax import lax
from jax.experimental import pallas as pl
from jax.experimental.pallas import tpu as pltpu
from jax.experimental.pallas import tpu_sc as plsc

_NUM_MOVABLE = 100000
_NUM_TERMINALS = 10000
_NUM_FILLER = 20000
_N = _NUM_MOVABLE + _NUM_TERMINALS + _NUM_FILLER
_NB = 512
_TARGET_DENSITY = 0.9
_SQRT2 = math.sqrt(2.0)
_K = 5

_NPAD = 131072
_CHUNK = _NPAD // 8
_SUB = 4096
_NSUB = _CHUNK // _SUB
_GSUB = _SUB // 16
_QROWS = 128
_TG0 = _NUM_MOVABLE // 16
_TG1 = (_NUM_MOVABLE + _NUM_TERMINALS) // 16


def _body(x_hbm, y_hbm, sx_hbm, sy_hbm, out_hbm, part_hbm,
          qmap, xb, yb, sxb, syb,
          acc, tmp0, tmp1, sem0, rsem):
    c = lax.axis_index("c")
    s = lax.axis_index("s")
    q = 2 * c + (s & 1)
    ql = s & 1
    chunk = s >> 1
    zeros16 = jnp.zeros((16,), jnp.float32)

    def zero_row(r, _):
        for j in range(_NB // 16):
            qmap[pl.ds(r * _NB + j * 16, 16)] = zeros16
        return 0
    lax.fori_loop(0, _QROWS, zero_row, 0)

    def scatter(ix0, iy0, wt, xe, ye, sxe, sye, ix0f, iy0f, kk):
        f = xe - ix0f
        fl = f + sxe
        g = ye - iy0f
        gl = g + sye
        px = [jnp.minimum(fl, 1.0) - f] + [
            jnp.maximum(jnp.minimum(fl - float(k), 1.0), 0.0) for k in range(1, kk)]
        py = [jnp.minimum(gl, 1.0) - g] + [
            jnp.maximum(jnp.minimum(gl - float(k), 1.0), 0.0) for k in range(1, kk)]
        lq = ix0 - (q << 7)
        wpx = [wt * p for p in px]
        rows = [lq + k for k in range(kk)]
        masks = [lax.bitcast_convert_type(r, jnp.uint32) < jnp.uint32(_QROWS)
                 for r in rows]
        py0 = jnp.where(iy0 >= 0, py[0], 0.0)
        pys = [py0] + py[1:]
        cols = [iy0 & (_NB - 1)] + [iy0 + k for k in range(1, kk)]
        rbase = [r << 9 for r in rows]
        for a in range(kk):
            for b in range(kk):
                plsc.addupdate_scatter(qmap, [rbase[a] + cols[b]], wpx[a] * pys[b],
                                       mask=masks[a])

    def make_body(terminal):
        def body(i, carry):
            sl = pl.ds(i * 16, 16)
            x = xb[sl]
            y = yb[sl]
            sx = sxb[sl]
            sy = syb[sl]
            if terminal:
                xe, ye, sxe, sye = x, y, sx, sy
                wt = jnp.full((16,), _TARGET_DENSITY, jnp.float32)
                ix0 = xe.astype(jnp.int32)
                iy0 = ye.astype(jnp.int32)
                kk = _K
            else:
                sxe = jnp.maximum(sx, _SQRT2)
                sye = jnp.maximum(sy, _SQRT2)
                xe = x + (sx - sxe) * 0.5
                ye = y + (sy - sye) * 0.5
                wt = (sx * sy) / (sxe * sye)
                ix0 = (xe + 1.0).astype(jnp.int32) - 1
                iy0 = (ye + 1.0).astype(jnp.int32) - 1
                kk = 3
            ix0f = ix0.astype(jnp.float32)
            iy0f = iy0.astype(jnp.float32)
            scatter(ix0, iy0, wt, xe, ye, sxe, sye, ix0f, iy0f, kk)
            return carry

        return body

    def process(glo, ghi, terminal):
        body = make_body(terminal)

        def win(wi, _):
            hsl = pl.ds(wi * _SUB, _SUB)
            cps = [pltpu.async_copy(x_hbm.at[hsl], xb, sem0),
                   pltpu.async_copy(y_hbm.at[hsl], yb, sem0),
                   pltpu.async_copy(sx_hbm.at[hsl], sxb, sem0),
                   pltpu.async_copy(sy_hbm.at[hsl], syb, sem0)]
            for cp in cps:
                cp.wait()
            base_g = wi * _GSUB
            lo = jnp.clip(glo - base_g, 0, _GSUB)
            hi = jnp.clip(ghi - base_g, 0, _GSUB)
            n = hi - lo

            def pair(j, carry):
                gg = lo + j * 2
                body(gg, 0)
                body(gg + 1, 0)
                return carry

            lax.fori_loop(0, n >> 1, pair, 0)

            @pl.when((n & 1) == 1)
            def _():
                body(hi - 1, 0)
            return 0

        lax.fori_loop(glo >> 8, (ghi + _GSUB - 1) >> 8, win, 0)

    ngroups = _NPAD // 16
    nstr = _TG0 + (ngroups - _TG1)
    v0 = (chunk * nstr) >> 3
    v1 = ((chunk + 1) * nstr) >> 3
    a0 = jnp.minimum(v0, _TG0)
    a1 = jnp.minimum(v1, _TG0)
    b0 = jnp.maximum(v0, _TG0) + (_TG1 - _TG0)
    b1 = jnp.maximum(v1, _TG0) + (_TG1 - _TG0)
    nterm = _TG1 - _TG0
    t0 = _TG0 + ((chunk * nterm) >> 3)
    t1 = _TG0 + (((chunk + 1) * nterm) >> 3)
    process(a0, a1, False)
    process(b0, b1, False)
    process(t0, t1, True)

    pltpu.sync_copy(qmap, part_hbm.at[c, s])
    plsc.subcore_barrier()

    p = s >> 1
    nred = _QROWS * _NB // 8
    rsl = pl.ds(p * nred, nred)
    pltpu.sync_copy(part_hbm.at[c, ql, rsl], acc)
    tmps = (tmp0, tmp1)
    cp = pltpu.async_copy(part_hbm.at[c, 2 + ql, rsl], tmp0, rsem)

    def add_tmp(t):
        def go(r, _):
            for j in range(8):
                csl = pl.ds(r * 128 + j * 16, 16)
                acc[csl] = acc[csl] + t[csl]
            return 0
        return go

    for k in range(1, 8):
        cp.wait()
        t = tmps[(k - 1) & 1]
        if k < 7:
            cp = pltpu.async_copy(part_hbm.at[c, 2 * (k + 1) + ql, rsl],
                                  tmps[k & 1], rsem)
        lax.fori_loop(0, nred // 128, add_tmp(t), 0)

    q0 = 2 * c + ql
    pltpu.sync_copy(acc, out_hbm.at[pl.ds(q0 * _QROWS * _NB + p * nred, nred)])


@jax.jit
def _density(xp, yp, sxp, syp):
    mesh = plsc.VectorSubcoreMesh(core_axis_name="c", subcore_axis_name="s")
    out, _ = pl.kernel(
        _body,
        out_type=(
            jax.ShapeDtypeStruct((_NB * _NB,), jnp.float32),
            jax.ShapeDtypeStruct((2, 16, _QROWS * _NB), jnp.float32),
        ),
        mesh=mesh,
        compiler_params=pltpu.CompilerParams(needs_layout_passes=False),
        scratch_types=[
            pltpu.VMEM((_QROWS * _NB,), jnp.float32),
            pltpu.VMEM((_SUB,), jnp.float32),
            pltpu.VMEM((_SUB,), jnp.float32),
            pltpu.VMEM((_SUB,), jnp.float32),
            pltpu.VMEM((_SUB,), jnp.float32),
            pltpu.VMEM((16 * _NB,), jnp.float32),
            pltpu.VMEM((16 * _NB,), jnp.float32),
            pltpu.VMEM((16 * _NB,), jnp.float32),
            pltpu.SemaphoreType.DMA,
            pltpu.SemaphoreType.DMA,
        ],
    )(xp, yp, sxp, syp)
    return out.reshape(_NB, _NB)


def kernel(pos, node_size_x, node_size_y):
    pad = _NPAD - _N
    xp = jnp.concatenate([pos[:_N], jnp.zeros((pad,), jnp.float32)])
    yp = jnp.concatenate([pos[_N:], jnp.zeros((pad,), jnp.float32)])
    sxp = jnp.concatenate([node_size_x, jnp.zeros((pad,), jnp.float32)])
    syp = jnp.concatenate([node_size_y, jnp.zeros((pad,), jnp.float32)])
    return _density(xp, yp, sxp, syp)


# speedup vs baseline: 1.0140x; 1.0140x over previous
import math

import jax
import jax.numpy as jnp
from jax import lax
from jax.experimental import pallas as pl
from jax.experimental.pallas import tpu as pltpu
from jax.experimental.pallas import tpu_sc as plsc

_NUM_MOVABLE = 100000
_NUM_TERMINALS = 10000
_NUM_FILLER = 20000
_N = _NUM_MOVABLE + _NUM_TERMINALS + _NUM_FILLER
_NB = 512
_TARGET_DENSITY = 0.9
_SQRT2 = math.sqrt(2.0)
_K = 5

_NPAD = 131072
_CHUNK = _NPAD // 8
_SUB = 4096
_NSUB = _CHUNK // _SUB
_GSUB = _SUB // 16
_QROWS = 128
_TG0 = _NUM_MOVABLE // 16
_TG1 = (_NUM_MOVABLE + _NUM_TERMINALS) // 16


def _body(x_hbm, y_hbm, sx_hbm, sy_hbm, out_hbm, part_hbm,
          qmap, xb, yb, sxb, syb,
          acc, tmp0, tmp1, sem0, rsem):
    c = lax.axis_index("c")
    s = lax.axis_index("s")
    q = 2 * c + (s & 1)
    ql = s & 1
    chunk = s >> 1
    zeros16 = jnp.zeros((16,), jnp.float32)

    def zero_row(r, _):
        for j in range(_NB // 16):
            qmap[pl.ds(r * _NB + j * 16, 16)] = zeros16
        return 0
    lax.fori_loop(0, _QROWS, zero_row, 0)

    def scatter(ix0, iy0, wt, xe, ye, sxe, sye, ix0f, iy0f, kk):
        f = xe - ix0f
        fl = f + sxe
        g = ye - iy0f
        gl = g + sye
        px = [jnp.minimum(fl, 1.0) - f] + [
            jnp.maximum(jnp.minimum(fl - float(k), 1.0), 0.0) for k in range(1, kk)]
        py = [jnp.minimum(gl, 1.0) - g] + [
            jnp.maximum(jnp.minimum(gl - float(k), 1.0), 0.0) for k in range(1, kk)]
        lq = ix0 - (q << 7)
        wpx = [wt * p for p in px]
        rows = [lq + k for k in range(kk)]
        masks = [lax.bitcast_convert_type(r, jnp.uint32) < jnp.uint32(_QROWS)
                 for r in rows]
        py0 = jnp.where(iy0 >= 0, py[0], 0.0)
        pys = [py0] + py[1:]
        cols = [iy0 & (_NB - 1)] + [iy0 + k for k in range(1, kk)]
        rbase = [r << 9 for r in rows]
        for a in range(kk):
            for b in range(kk):
                plsc.addupdate_scatter(qmap, [rbase[a] + cols[b]], wpx[a] * pys[b],
                                       mask=masks[a])

    def make_body(terminal):
        def body(i, carry):
            sl = pl.ds(i * 16, 16)
            x = xb[sl]
            y = yb[sl]
            sx = sxb[sl]
            sy = syb[sl]
            if terminal:
                xe, ye, sxe, sye = x, y, sx, sy
                wt = jnp.full((16,), _TARGET_DENSITY, jnp.float32)
                ix0 = xe.astype(jnp.int32)
                iy0 = ye.astype(jnp.int32)
                kk = _K
            else:
                sxe = jnp.maximum(sx, _SQRT2)
                sye = jnp.maximum(sy, _SQRT2)
                xe = x + (sx - sxe) * 0.5
                ye = y + (sy - sye) * 0.5
                wt = (sx * sy) / (sxe * sye)
                ix0 = (xe + 1.0).astype(jnp.int32) - 1
                iy0 = (ye + 1.0).astype(jnp.int32) - 1
                kk = 3
            ix0f = ix0.astype(jnp.float32)
            iy0f = iy0.astype(jnp.float32)
            scatter(ix0, iy0, wt, xe, ye, sxe, sye, ix0f, iy0f, kk)
            return carry

        return body

    def process(glo, ghi, terminal):
        body = make_body(terminal)

        def win(wi, _):
            hsl = pl.ds(wi * _SUB, _SUB)
            cps = [pltpu.async_copy(x_hbm.at[hsl], xb, sem0),
                   pltpu.async_copy(y_hbm.at[hsl], yb, sem0),
                   pltpu.async_copy(sx_hbm.at[hsl], sxb, sem0),
                   pltpu.async_copy(sy_hbm.at[hsl], syb, sem0)]
            for cp in cps:
                cp.wait()
            base_g = wi * _GSUB
            lo = jnp.clip(glo - base_g, 0, _GSUB)
            hi = jnp.clip(ghi - base_g, 0, _GSUB)
            lax.fori_loop(lo, hi, body, 0)
            return 0

        lax.fori_loop(glo >> 8, (ghi + _GSUB - 1) >> 8, win, 0)

    ngroups = _NPAD // 16
    nstr = _TG0 + (ngroups - _TG1)
    v0 = (chunk * nstr) >> 3
    v1 = ((chunk + 1) * nstr) >> 3
    a0 = jnp.minimum(v0, _TG0)
    a1 = jnp.minimum(v1, _TG0)
    b0 = jnp.maximum(v0, _TG0) + (_TG1 - _TG0)
    b1 = jnp.maximum(v1, _TG0) + (_TG1 - _TG0)
    nterm = _TG1 - _TG0
    t0 = _TG0 + ((chunk * nterm) >> 3)
    t1 = _TG0 + (((chunk + 1) * nterm) >> 3)
    process(a0, a1, False)
    process(b0, b1, False)
    process(t0, t1, True)

    pltpu.sync_copy(qmap, part_hbm.at[c, s])
    plsc.subcore_barrier()

    p = s >> 1
    nred = _QROWS * _NB // 8
    rsl = pl.ds(p * nred, nred)
    pltpu.sync_copy(part_hbm.at[c, ql, rsl], acc)
    tmps = (tmp0, tmp1)
    cp = pltpu.async_copy(part_hbm.at[c, 2 + ql, rsl], tmp0, rsem)

    def add_tmp(t):
        def go(r, _):
            for j in range(8):
                csl = pl.ds(r * 128 + j * 16, 16)
                acc[csl] = acc[csl] + t[csl]
            return 0
        return go

    for k in range(1, 8):
        cp.wait()
        t = tmps[(k - 1) & 1]
        if k < 7:
            cp = pltpu.async_copy(part_hbm.at[c, 2 * (k + 1) + ql, rsl],
                                  tmps[k & 1], rsem)
        lax.fori_loop(0, nred // 128, add_tmp(t), 0)

    q0 = 2 * c + ql
    pltpu.sync_copy(acc, out_hbm.at[pl.ds(q0 * _QROWS * _NB + p * nred, nred)])


@jax.jit
def _density(xp, yp, sxp, syp):
    mesh = plsc.VectorSubcoreMesh(core_axis_name="c", subcore_axis_name="s")
    out, _ = pl.kernel(
        _body,
        out_type=(
            jax.ShapeDtypeStruct((_NB * _NB,), jnp.float32),
            jax.ShapeDtypeStruct((2, 16, _QROWS * _NB), jnp.float32),
        ),
        mesh=mesh,
        compiler_params=pltpu.CompilerParams(needs_layout_passes=False),
        scratch_types=[
            pltpu.VMEM((_QROWS * _NB,), jnp.float32),
            pltpu.VMEM((_SUB,), jnp.float32),
            pltpu.VMEM((_SUB,), jnp.float32),
            pltpu.VMEM((_SUB,), jnp.float32),
            pltpu.VMEM((_SUB,), jnp.float32),
            pltpu.VMEM((16 * _NB,), jnp.float32),
            pltpu.VMEM((16 * _NB,), jnp.float32),
            pltpu.VMEM((16 * _NB,), jnp.float32),
            pltpu.SemaphoreType.DMA,
            pltpu.SemaphoreType.DMA,
        ],
    )(xp, yp, sxp, syp)
    return out.reshape(_NB, _NB)


def kernel(pos, node_size_x, node_size_y):
    pad = _NPAD - _N
    xp = jnp.concatenate([pos[:_N], jnp.zeros((pad,), jnp.float32)])
    yp = jnp.concatenate([pos[_N:], jnp.zeros((pad,), jnp.float32)])
    sxp = jnp.concatenate([node_size_x, jnp.zeros((pad,), jnp.float32)])
    syp = jnp.concatenate([node_size_y, jnp.zeros((pad,), jnp.float32)])
    return _density(xp, yp, sxp, syp)


# speedup vs baseline: 1.0907x; 1.0756x over previous
import math

import jax
import jax.numpy as jnp
from jax import lax
from jax.experimental import pallas as pl
from jax.experimental.pallas import tpu as pltpu
from jax.experimental.pallas import tpu_sc as plsc

_NUM_MOVABLE = 100000
_NUM_TERMINALS = 10000
_NUM_FILLER = 20000
_N = _NUM_MOVABLE + _NUM_TERMINALS + _NUM_FILLER
_NB = 512
_TARGET_DENSITY = 0.9
_SQRT2 = math.sqrt(2.0)
_K = 5

_SUB = 4096
_GSUB = _SUB // 16
_QROWS = 128
_TG0 = _NUM_MOVABLE // 16
_TG1 = (_NUM_MOVABLE + _NUM_TERMINALS) // 16
_NGROUPS = _N // 16
_LASTW = (_NGROUPS - 1) // _GSUB
_TAILC = _N - _LASTW * _SUB
_YSH = 8


def _body(pos_hbm, sx_hbm, sy_hbm, out_hbm, part_hbm,
          qmap, xb, yb, sxb, syb,
          acc, tmp0, tmp1, sem0, rsem):
    c = lax.axis_index("c")
    s = lax.axis_index("s")
    q = 2 * c + (s & 1)
    ql = s & 1
    chunk = s >> 1
    zeros16 = jnp.zeros((16,), jnp.float32)

    def zero_row(r, _):
        for j in range(_NB // 16):
            qmap[pl.ds(r * _NB + j * 16, 16)] = zeros16
        return 0
    lax.fori_loop(0, _QROWS, zero_row, 0)

    def scatter(ix0, iy0, wt, xe, ye, sxe, sye, ix0f, iy0f, kk):
        f = xe - ix0f
        fl = f + sxe
        g = ye - iy0f
        gl = g + sye
        px = [jnp.minimum(fl, 1.0) - f] + [
            jnp.maximum(jnp.minimum(fl - float(k), 1.0), 0.0) for k in range(1, kk)]
        py = [jnp.minimum(gl, 1.0) - g] + [
            jnp.maximum(jnp.minimum(gl - float(k), 1.0), 0.0) for k in range(1, kk)]
        lq = ix0 - (q << 7)
        wpx = [wt * p for p in px]
        rows = [lq + k for k in range(kk)]
        masks = [lax.bitcast_convert_type(r, jnp.uint32) < jnp.uint32(_QROWS)
                 for r in rows]
        py0 = jnp.where(iy0 >= 0, py[0], 0.0)
        pys = [py0] + py[1:]
        cols = [iy0 & (_NB - 1)] + [iy0 + k for k in range(1, kk)]
        rbase = [r << 9 for r in rows]
        for a in range(kk):
            for b in range(kk):
                plsc.addupdate_scatter(qmap, [rbase[a] + cols[b]], wpx[a] * pys[b],
                                       mask=masks[a])

    def make_body(terminal):
        def body(i, carry):
            sl = pl.ds(i * 16, 16)
            x = xb[sl]
            y = yb[pl.ds(i * 16 + _YSH, 16)]
            sx = sxb[sl]
            sy = syb[sl]
            if terminal:
                xe, ye, sxe, sye = x, y, sx, sy
                wt = jnp.full((16,), _TARGET_DENSITY, jnp.float32)
                ix0 = xe.astype(jnp.int32)
                iy0 = ye.astype(jnp.int32)
                kk = _K
            else:
                sxe = jnp.maximum(sx, _SQRT2)
                sye = jnp.maximum(sy, _SQRT2)
                xe = x + (sx - sxe) * 0.5
                ye = y + (sy - sye) * 0.5
                wt = (sx * sy) / (sxe * sye)
                ix0 = (xe + 1.0).astype(jnp.int32) - 1
                iy0 = (ye + 1.0).astype(jnp.int32) - 1
                kk = 3
            ix0f = ix0.astype(jnp.float32)
            iy0f = iy0.astype(jnp.float32)
            scatter(ix0, iy0, wt, xe, ye, sxe, sye, ix0f, iy0f, kk)
            return carry

        return body

    def process(glo, ghi, terminal):
        body = make_body(terminal)

        def win(wi, _):
            xbase = wi * _SUB
            ybase = _N - _YSH + wi * _SUB

            @pl.when(wi < _LASTW)
            def _():
                for cp in [
                    pltpu.async_copy(pos_hbm.at[pl.ds(xbase, _SUB)], xb, sem0),
                    pltpu.async_copy(pos_hbm.at[pl.ds(ybase, _SUB + 2 * _YSH)], yb, sem0),
                    pltpu.async_copy(sx_hbm.at[pl.ds(xbase, _SUB)], sxb, sem0),
                    pltpu.async_copy(sy_hbm.at[pl.ds(xbase, _SUB)], syb, sem0),
                ]:
                    cp.wait()

            @pl.when(wi == _LASTW)
            def _():
                for cp in [
                    pltpu.async_copy(pos_hbm.at[pl.ds(xbase, _TAILC)],
                                     xb.at[pl.ds(0, _TAILC)], sem0),
                    pltpu.async_copy(pos_hbm.at[pl.ds(ybase, _TAILC + _YSH)],
                                     yb.at[pl.ds(0, _TAILC + _YSH)], sem0),
                    pltpu.async_copy(sx_hbm.at[pl.ds(xbase, _TAILC)],
                                     sxb.at[pl.ds(0, _TAILC)], sem0),
                    pltpu.async_copy(sy_hbm.at[pl.ds(xbase, _TAILC)],
                                     syb.at[pl.ds(0, _TAILC)], sem0),
                ]:
                    cp.wait()
            base_g = wi * _GSUB
            lo = jnp.clip(glo - base_g, 0, _GSUB)
            hi = jnp.clip(ghi - base_g, 0, _GSUB)
            lax.fori_loop(lo, hi, body, 0)
            return 0

        lax.fori_loop(glo >> 8, (ghi + _GSUB - 1) >> 8, win, 0)

    ngroups = _NGROUPS
    nstr = _TG0 + (ngroups - _TG1)
    v0 = (chunk * nstr) >> 3
    v1 = ((chunk + 1) * nstr) >> 3
    a0 = jnp.minimum(v0, _TG0)
    a1 = jnp.minimum(v1, _TG0)
    b0 = jnp.maximum(v0, _TG0) + (_TG1 - _TG0)
    b1 = jnp.maximum(v1, _TG0) + (_TG1 - _TG0)
    nterm = _TG1 - _TG0
    t0 = _TG0 + ((chunk * nterm) >> 3)
    t1 = _TG0 + (((chunk + 1) * nterm) >> 3)
    process(a0, a1, False)
    process(b0, b1, False)
    process(t0, t1, True)

    pltpu.sync_copy(qmap, part_hbm.at[c, s])
    plsc.subcore_barrier()

    p = s >> 1
    nred = _QROWS * _NB // 8
    rsl = pl.ds(p * nred, nred)
    pltpu.sync_copy(part_hbm.at[c, ql, rsl], acc)
    tmps = (tmp0, tmp1)
    cp = pltpu.async_copy(part_hbm.at[c, 2 + ql, rsl], tmp0, rsem)

    def add_tmp(t):
        def go(r, _):
            for j in range(8):
                csl = pl.ds(r * 128 + j * 16, 16)
                acc[csl] = acc[csl] + t[csl]
            return 0
        return go

    for k in range(1, 8):
        cp.wait()
        t = tmps[(k - 1) & 1]
        if k < 7:
            cp = pltpu.async_copy(part_hbm.at[c, 2 * (k + 1) + ql, rsl],
                                  tmps[k & 1], rsem)
        lax.fori_loop(0, nred // 128, add_tmp(t), 0)

    q0 = 2 * c + ql
    pltpu.sync_copy(acc, out_hbm.at[pl.ds(q0 * _QROWS * _NB + p * nred, nred)])


@jax.jit
def _density(pos, sxp, syp):
    mesh = plsc.VectorSubcoreMesh(core_axis_name="c", subcore_axis_name="s")
    out, _ = pl.kernel(
        _body,
        out_type=(
            jax.ShapeDtypeStruct((_NB * _NB,), jnp.float32),
            jax.ShapeDtypeStruct((2, 16, _QROWS * _NB), jnp.float32),
        ),
        mesh=mesh,
        compiler_params=pltpu.CompilerParams(needs_layout_passes=False),
        scratch_types=[
            pltpu.VMEM((_QROWS * _NB,), jnp.float32),
            pltpu.VMEM((_SUB,), jnp.float32),
            pltpu.VMEM((_SUB + 2 * _YSH,), jnp.float32),
            pltpu.VMEM((_SUB,), jnp.float32),
            pltpu.VMEM((_SUB,), jnp.float32),
            pltpu.VMEM((16 * _NB,), jnp.float32),
            pltpu.VMEM((16 * _NB,), jnp.float32),
            pltpu.VMEM((16 * _NB,), jnp.float32),
            pltpu.SemaphoreType.DMA,
            pltpu.SemaphoreType.DMA,
        ],
    )(pos, sxp, syp)
    return out.reshape(_NB, _NB)


def kernel(pos, node_size_x, node_size_y):
    return _density(pos, node_size_x, node_size_y)


# speedup vs baseline: 1.0927x; 1.0019x over previous
import math

import jax
import jax.numpy as jnp
from jax import lax
from jax.experimental import pallas as pl
from jax.experimental.pallas import tpu as pltpu
from jax.experimental.pallas import tpu_sc as plsc

_NUM_MOVABLE = 100000
_NUM_TERMINALS = 10000
_NUM_FILLER = 20000
_N = _NUM_MOVABLE + _NUM_TERMINALS + _NUM_FILLER
_NB = 512
_TARGET_DENSITY = 0.9
_SQRT2 = math.sqrt(2.0)
_K = 5

_SUB = 8192
_GSUB = _SUB // 16
_QROWS = 128
_TG0 = _NUM_MOVABLE // 16
_TG1 = (_NUM_MOVABLE + _NUM_TERMINALS) // 16
_NGROUPS = _N // 16
_LASTW = (_NGROUPS - 1) // _GSUB
_TAILC = _N - _LASTW * _SUB
_YSH = 8


def _body(pos_hbm, sx_hbm, sy_hbm, out_hbm, part_hbm,
          qmap, xb, yb, sxb, syb,
          acc, tmp0, tmp1, sem0, rsem):
    c = lax.axis_index("c")
    s = lax.axis_index("s")
    q = 2 * c + (s & 1)
    ql = s & 1
    chunk = s >> 1
    zeros16 = jnp.zeros((16,), jnp.float32)

    def zero_row(r, _):
        for j in range(_NB // 16):
            qmap[pl.ds(r * _NB + j * 16, 16)] = zeros16
        return 0
    lax.fori_loop(0, _QROWS, zero_row, 0)

    def scatter(ix0, iy0, wt, xe, ye, sxe, sye, ix0f, iy0f, kk):
        f = xe - ix0f
        fl = f + sxe
        g = ye - iy0f
        gl = g + sye
        px = [jnp.minimum(fl, 1.0) - f] + [
            jnp.maximum(jnp.minimum(fl - float(k), 1.0), 0.0) for k in range(1, kk)]
        py = [jnp.minimum(gl, 1.0) - g] + [
            jnp.maximum(jnp.minimum(gl - float(k), 1.0), 0.0) for k in range(1, kk)]
        lq = ix0 - (q << 7)
        wpx = [wt * p for p in px]
        rows = [lq + k for k in range(kk)]
        masks = [lax.bitcast_convert_type(r, jnp.uint32) < jnp.uint32(_QROWS)
                 for r in rows]
        py0 = jnp.where(iy0 >= 0, py[0], 0.0)
        pys = [py0] + py[1:]
        cols = [iy0 & (_NB - 1)] + [iy0 + k for k in range(1, kk)]
        rbase = [r << 9 for r in rows]
        for a in range(kk):
            for b in range(kk):
                plsc.addupdate_scatter(qmap, [rbase[a] + cols[b]], wpx[a] * pys[b],
                                       mask=masks[a])

    def make_body(terminal):
        def body(i, carry):
            sl = pl.ds(i * 16, 16)
            x = xb[sl]
            y = yb[pl.ds(i * 16 + _YSH, 16)]
            sx = sxb[sl]
            sy = syb[sl]
            if terminal:
                xe, ye, sxe, sye = x, y, sx, sy
                wt = jnp.full((16,), _TARGET_DENSITY, jnp.float32)
                ix0 = xe.astype(jnp.int32)
                iy0 = ye.astype(jnp.int32)
                kk = _K
            else:
                sxe = jnp.maximum(sx, _SQRT2)
                sye = jnp.maximum(sy, _SQRT2)
                xe = x + (sx - sxe) * 0.5
                ye = y + (sy - sye) * 0.5
                wt = (sx * sy) / (sxe * sye)
                ix0 = (xe + 1.0).astype(jnp.int32) - 1
                iy0 = (ye + 1.0).astype(jnp.int32) - 1
                kk = 3
            ix0f = ix0.astype(jnp.float32)
            iy0f = iy0.astype(jnp.float32)
            scatter(ix0, iy0, wt, xe, ye, sxe, sye, ix0f, iy0f, kk)
            return carry

        return body

    def process(glo, ghi, terminal):
        body = make_body(terminal)

        def win(wi, _):
            xbase = wi * _SUB
            ybase = _N - _YSH + wi * _SUB

            @pl.when(wi < _LASTW)
            def _():
                for cp in [
                    pltpu.async_copy(pos_hbm.at[pl.ds(xbase, _SUB)], xb, sem0),
                    pltpu.async_copy(pos_hbm.at[pl.ds(ybase, _SUB + 2 * _YSH)], yb, sem0),
                    pltpu.async_copy(sx_hbm.at[pl.ds(xbase, _SUB)], sxb, sem0),
                    pltpu.async_copy(sy_hbm.at[pl.ds(xbase, _SUB)], syb, sem0),
                ]:
                    cp.wait()

            @pl.when(wi == _LASTW)
            def _():
                for cp in [
                    pltpu.async_copy(pos_hbm.at[pl.ds(xbase, _TAILC)],
                                     xb.at[pl.ds(0, _TAILC)], sem0),
                    pltpu.async_copy(pos_hbm.at[pl.ds(ybase, _TAILC + _YSH)],
                                     yb.at[pl.ds(0, _TAILC + _YSH)], sem0),
                    pltpu.async_copy(sx_hbm.at[pl.ds(xbase, _TAILC)],
                                     sxb.at[pl.ds(0, _TAILC)], sem0),
                    pltpu.async_copy(sy_hbm.at[pl.ds(xbase, _TAILC)],
                                     syb.at[pl.ds(0, _TAILC)], sem0),
                ]:
                    cp.wait()
            base_g = wi * _GSUB
            lo = jnp.clip(glo - base_g, 0, _GSUB)
            hi = jnp.clip(ghi - base_g, 0, _GSUB)
            lax.fori_loop(lo, hi, body, 0)
            return 0

        lax.fori_loop(glo // _GSUB, (ghi + _GSUB - 1) // _GSUB, win, 0)

    ngroups = _NGROUPS
    nstr = _TG0 + (ngroups - _TG1)
    v0 = (chunk * nstr) >> 3
    v1 = ((chunk + 1) * nstr) >> 3
    a0 = jnp.minimum(v0, _TG0)
    a1 = jnp.minimum(v1, _TG0)
    b0 = jnp.maximum(v0, _TG0) + (_TG1 - _TG0)
    b1 = jnp.maximum(v1, _TG0) + (_TG1 - _TG0)
    nterm = _TG1 - _TG0
    t0 = _TG0 + ((chunk * nterm) >> 3)
    t1 = _TG0 + (((chunk + 1) * nterm) >> 3)
    process(a0, a1, False)
    process(b0, b1, False)
    process(t0, t1, True)

    pltpu.sync_copy(qmap, part_hbm.at[c, s])
    plsc.subcore_barrier()

    p = s >> 1
    nred = _QROWS * _NB // 8
    rsl = pl.ds(p * nred, nred)
    pltpu.sync_copy(part_hbm.at[c, ql, rsl], acc)
    tmps = (tmp0, tmp1)
    cp = pltpu.async_copy(part_hbm.at[c, 2 + ql, rsl], tmp0, rsem)

    def add_tmp(t):
        def go(r, _):
            for j in range(8):
                csl = pl.ds(r * 128 + j * 16, 16)
                acc[csl] = acc[csl] + t[csl]
            return 0
        return go

    for k in range(1, 8):
        cp.wait()
        t = tmps[(k - 1) & 1]
        if k < 7:
            cp = pltpu.async_copy(part_hbm.at[c, 2 * (k + 1) + ql, rsl],
                                  tmps[k & 1], rsem)
        lax.fori_loop(0, nred // 128, add_tmp(t), 0)

    q0 = 2 * c + ql
    pltpu.sync_copy(acc, out_hbm.at[pl.ds(q0 * _QROWS * _NB + p * nred, nred)])


@jax.jit
def _density(pos, sxp, syp):
    mesh = plsc.VectorSubcoreMesh(core_axis_name="c", subcore_axis_name="s")
    out, _ = pl.kernel(
        _body,
        out_type=(
            jax.ShapeDtypeStruct((_NB * _NB,), jnp.float32),
            jax.ShapeDtypeStruct((2, 16, _QROWS * _NB), jnp.float32),
        ),
        mesh=mesh,
        compiler_params=pltpu.CompilerParams(needs_layout_passes=False),
        scratch_types=[
            pltpu.VMEM((_QROWS * _NB,), jnp.float32),
            pltpu.VMEM((_SUB,), jnp.float32),
            pltpu.VMEM((_SUB + 2 * _YSH,), jnp.float32),
            pltpu.VMEM((_SUB,), jnp.float32),
            pltpu.VMEM((_SUB,), jnp.float32),
            pltpu.VMEM((16 * _NB,), jnp.float32),
            pltpu.VMEM((16 * _NB,), jnp.float32),
            pltpu.VMEM((16 * _NB,), jnp.float32),
            pltpu.SemaphoreType.DMA,
            pltpu.SemaphoreType.DMA,
        ],
    )(pos, sxp, syp)
    return out.reshape(_NB, _NB)


def kernel(pos, node_size_x, node_size_y):
    return _density(pos, node_size_x, node_size_y)
